# baseline (speedup 1.0000x reference)
import jax
import jax.numpy as jnp
from jax.experimental import pallas as pl
from jax.experimental.pallas import tpu as pltpu


def _fused_kernel(x_ref, w_ref, s0_ref, s1_ref, o_ref, xw_ref):
    @pl.when(pl.program_id(0) == 0)
    def _():
        ba = jnp.where(x_ref[:] > 0, 1.0, 0.0).astype(jnp.bfloat16)
        bw = jnp.sign(w_ref[:]).astype(jnp.bfloat16)
        acc = jax.lax.dot_general(
            ba, bw, (((1,), (1,)), ((), ())),
            preferred_element_type=jnp.float32,
        )
        xw_ref[:] = acc.astype(jnp.bfloat16)

    dn = (((1,), (0,)), ((), ()))
    acc0 = jax.lax.dot_general(
        s0_ref[:].astype(jnp.bfloat16), xw_ref[:],
        dn, preferred_element_type=jnp.float32,
    )
    acc1 = jax.lax.dot_general(
        s1_ref[:].astype(jnp.bfloat16), xw_ref[:],
        dn, preferred_element_type=jnp.float32,
    )
    o_ref[0] = jnp.maximum(acc0, 0.0)
    o_ref[1] = jnp.maximum(acc1, 0.0)


def kernel(x, support, W):
    n, d_in = x.shape
    d_out = W.shape[0]
    tm = 200
    nblk = n // (2 * tm)

    out = pl.pallas_call(
        _fused_kernel,
        grid=(nblk,),
        in_specs=[
            pl.BlockSpec((n, d_in), lambda i: (0, 0)),
            pl.BlockSpec((d_out, d_in), lambda i: (0, 0)),
            pl.BlockSpec((tm, n), lambda i: (i, 0)),
            pl.BlockSpec((tm, n), lambda i: (i + nblk, 0)),
        ],
        out_specs=pl.BlockSpec((2, tm, d_out), lambda i: (0, i, 0)),
        out_shape=jax.ShapeDtypeStruct((2, n // 2, d_out), jnp.float32),
        scratch_shapes=[pltpu.VMEM((n, d_out), jnp.bfloat16)],
        compiler_params=pltpu.CompilerParams(
            dimension_semantics=("arbitrary",),
        ),
    )(x, W, support, support)

    return (out.reshape(n, d_out), support)


# speedup vs baseline: 1.0029x; 1.0029x over previous
import jax
import jax.numpy as jnp
from jax.experimental import pallas as pl
from jax.experimental.pallas import tpu as pltpu


def _fused_kernel(x_ref, w_ref, s_ref, o_ref, xw_ref):
    @pl.when(pl.program_id(0) == 0)
    def _():
        ba = jnp.where(x_ref[:] > 0, 1.0, 0.0).astype(jnp.bfloat16)
        bw = jnp.sign(w_ref[:]).astype(jnp.bfloat16)
        acc = jax.lax.dot_general(
            ba, bw, (((1,), (1,)), ((), ())),
            preferred_element_type=jnp.float32,
        )
        xw_ref[:] = acc.astype(jnp.bfloat16)

    sb = s_ref[:].astype(jnp.bfloat16)
    acc = jax.lax.dot_general(
        sb, xw_ref[:], (((1,), (0,)), ((), ())),
        preferred_element_type=jnp.float32,
    )
    o_ref[:] = jnp.maximum(acc, 0.0)


def kernel(x, support, W):
    n, d_in = x.shape
    d_out = W.shape[0]
    tm = 200

    out = pl.pallas_call(
        _fused_kernel,
        grid=(n // tm,),
        in_specs=[
            pl.BlockSpec((n, d_in), lambda i: (0, 0)),
            pl.BlockSpec((d_out, d_in), lambda i: (0, 0)),
            pl.BlockSpec((tm, n), lambda i: (i, 0)),
        ],
        out_specs=pl.BlockSpec((tm, d_out), lambda i: (i, 0)),
        out_shape=jax.ShapeDtypeStruct((n, d_out), jnp.float32),
        scratch_shapes=[pltpu.VMEM((n, d_out), jnp.bfloat16)],
        compiler_params=pltpu.CompilerParams(
            dimension_semantics=("arbitrary",),
        ),
    )(x, W, support)

    return (out, support)
